# baseline (speedup 1.0000x reference)
import functools

import jax
import jax.numpy as jnp
from jax import lax
from jax.experimental import pallas as pl
from jax.experimental.pallas import tpu as pltpu
from jax.experimental.pallas import tpu_sc as plsc

D1, D2, ODIM = 1000, 1000, 64
BATCH = 16384

NC, NS, L = 2, 16, 16
NW = NC * NS
BPW = BATCH // NW
CHUNK = 128
NCHUNK = BPW // CHUNK

_mesh = plsc.VectorSubcoreMesh(core_axis_name="c", subcore_axis_name="s")


@functools.partial(
    pl.kernel,
    mesh=_mesh,
    out_type=jax.ShapeDtypeStruct((BATCH, ODIM), jnp.float32),
    compiler_params=pltpu.CompilerParams(use_tc_tiling_on_sc=False),
    scratch_types=[
        pltpu.VMEM((BPW,), jnp.int32),
        pltpu.VMEM((BPW,), jnp.int32),
        pltpu.VMEM((BPW,), jnp.int32),
        pltpu.VMEM((BPW, ODIM), jnp.float32),
        pltpu.SemaphoreType.DMA,
    ],
)
def _gather_kernel(c0_hbm, c1_hbm, table_hbm, out_hbm,
                   c0_v, c1_v, idx_v, rows_v, sem):
    wid = lax.axis_index("s") * NC + lax.axis_index("c")
    base = wid * BPW
    pltpu.sync_copy(c0_hbm.at[pl.ds(base, BPW)], c0_v)
    pltpu.sync_copy(c1_hbm.at[pl.ds(base, BPW)], c1_v)
    for i in range(BPW // L):
        s = pl.ds(i * L, L)
        idx_v[s] = c0_v[s] + c1_v[s] * D1
    copies = [
        pltpu.async_copy(
            table_hbm.at[idx_v.at[pl.ds(j * CHUNK, CHUNK)]],
            rows_v.at[pl.ds(j * CHUNK, CHUNK)],
            sem,
        )
        for j in range(NCHUNK)
    ]
    for cp in copies:
        cp.wait()
    pltpu.sync_copy(rows_v, out_hbm.at[pl.ds(base, BPW)])


def kernel(code0, code1, embeddings):
    table = embeddings.reshape(-1, ODIM)
    c0 = code0.astype(jnp.int32)
    c1 = code1.astype(jnp.int32)
    return _gather_kernel(c0, c1, table)


# speedup vs baseline: 1.6193x; 1.6193x over previous
import functools

import jax
import jax.numpy as jnp
from jax import lax
from jax.experimental import pallas as pl
from jax.experimental.pallas import tpu as pltpu
from jax.experimental.pallas import tpu_sc as plsc

D1, D2, ODIM = 1000, 1000, 64
BATCH = 16384

NC, NS = 2, 16
NW = NC * NS
BPW = BATCH // NW

_mesh = plsc.VectorSubcoreMesh(core_axis_name="c", subcore_axis_name="s")


@functools.partial(
    pl.kernel,
    mesh=_mesh,
    out_type=jax.ShapeDtypeStruct((BATCH, ODIM), jnp.float32),
    scratch_types=[
        pltpu.VMEM((BPW,), jnp.int32),
        pltpu.VMEM((BPW,), jnp.int32),
        pltpu.VMEM((BPW, ODIM), jnp.float32),
        pltpu.SemaphoreType.DMA,
    ],
)
def _gather_kernel(c0_hbm, c1_hbm, emb_hbm, out_hbm, c0_v, c1_v, rows_v, sem):
    wid = lax.axis_index("s") * NC + lax.axis_index("c")
    base = wid * BPW
    pltpu.sync_copy(c0_hbm.at[pl.ds(base, BPW)], c0_v)
    pltpu.sync_copy(c1_hbm.at[pl.ds(base, BPW)], c1_v)

    def issue(g, _):
        g16 = g * jnp.int32(16)
        s = pl.ds(g16, 16)
        c0g = c0_v[s]
        c1g = c1_v[s]
        for k in range(16):
            pltpu.async_copy(
                emb_hbm.at[c1g[k], c0g[k]], rows_v.at[g16 + jnp.int32(k)], sem
            )
        return 0

    lax.fori_loop(jnp.int32(0), jnp.int32(BPW // 16), issue, 0)
    pltpu.make_async_copy(out_hbm.at[pl.ds(base, BPW)], rows_v, sem).wait()
    pltpu.sync_copy(rows_v, out_hbm.at[pl.ds(base, BPW)])


def kernel(code0, code1, embeddings):
    c0 = code0.astype(jnp.int32)
    c1 = code1.astype(jnp.int32)
    return _gather_kernel(c0, c1, embeddings)
